# baseline (speedup 1.0000x reference)
import functools
import math

import jax
import jax.numpy as jnp
from jax import lax
from jax.experimental import pallas as pl
from jax.experimental.pallas import tpu as pltpu
from jax.experimental.pallas import tpu_sc as plsc

_D = 64
_SCALE = 8.0
_LANES = 16
_NDMA = 8


@functools.cache
def _make_sc_gather(B, V, D, chunk):
    NC, NS = 2, 16
    NW = NC * NS
    b_per_w = B // NW
    assert b_per_w * NW == B and b_per_w % chunk == 0
    n_chunks = b_per_w // chunk
    mesh = plsc.VectorSubcoreMesh(core_axis_name="c", subcore_axis_name="s")

    @functools.partial(
        pl.kernel,
        out_type=jax.ShapeDtypeStruct((B, D), jnp.float32),
        mesh=mesh,
        scratch_types=[
            pltpu.VMEM((chunk,), jnp.int32),
            pltpu.VMEM((chunk,), jnp.int32),
            pltpu.VMEM((chunk, D), jnp.float32),
            pltpu.VMEM((chunk, D), jnp.float32),
            pltpu.SemaphoreType.DMA,
            pltpu.SemaphoreType.DMA,
            pltpu.SemaphoreType.DMA,
            pltpu.SemaphoreType.DMA,
        ],
        compiler_params=pltpu.CompilerParams(use_tc_tiling_on_sc=False),
    )
    def sc_gather(x_hbm, table_hbm, out_hbm, idx0, idx1, rows0, rows1,
                  sg0, sg1, ss0, ss1):
        wid = lax.axis_index("s") * NC + lax.axis_index("c")
        base = wid * b_per_w
        slots = ((idx0, rows0, sg0, ss0), (idx1, rows1, sg1, ss1))

        def start_gather(g, slot):
            idx, rows, sg, _ = slot
            pltpu.sync_copy(x_hbm.at[pl.ds(base + g * chunk, chunk)], idx)
            pltpu.async_copy(table_hbm.at[idx], rows, sg)

        def wait_gather(slot):
            idx, rows, sg, _ = slot
            pltpu.make_async_copy(table_hbm.at[idx], rows, sg).wait()

        def scale(slot):
            rows = slot[1]

            def row_body(r, _):
                for j in range(D // _LANES):
                    sl = pl.ds(j * _LANES, _LANES)
                    rows[r, sl] = rows[r, sl] * _SCALE
                return ()

            lax.fori_loop(0, chunk, row_body, (), unroll=8)

        def start_scatter(g, slot):
            _, rows, _, ss = slot
            pltpu.async_copy(rows, out_hbm.at[pl.ds(base + g * chunk, chunk)], ss)

        def wait_scatter(g, slot):
            _, rows, _, ss = slot
            pltpu.make_async_copy(
                rows, out_hbm.at[pl.ds(base + g * chunk, chunk)], ss).wait()

        start_gather(0, slots[0])

        def pair(p, _):
            for b in range(2):
                g = p * 2 + b
                nslot = slots[1 - b]

                @pl.when(g + 1 < n_chunks)
                def _():
                    @pl.when(g >= 1)
                    def _():
                        wait_scatter(g - 1, nslot)

                    start_gather(g + 1, nslot)

                wait_gather(slots[b])
                scale(slots[b])
                start_scatter(g, slots[b])
            return ()

        lax.fori_loop(0, n_chunks // 2, pair, ())
        wait_scatter(n_chunks - 2, slots[0])
        wait_scatter(n_chunks - 1, slots[1])

    return sc_gather


@functools.cache
def _make_tc_relayout(S0, S1, D):
    B = S0 * S1
    rows = S0 // _NDMA

    def body(i_ref, o_ref, sem):
        i3 = i_ref.reshape(S0, S1, D)
        copies = [
            pltpu.make_async_copy(
                i3.at[pl.ds(k * rows, rows)],
                o_ref.at[pl.ds(k * rows, rows)],
                sem.at[k],
            )
            for k in range(_NDMA)
        ]
        for cp in copies:
            cp.start()
        for cp in copies:
            cp.wait()

    return pl.pallas_call(
        body,
        in_specs=[pl.BlockSpec(memory_space=pl.ANY)],
        out_specs=pl.BlockSpec(memory_space=pl.ANY),
        out_shape=jax.ShapeDtypeStruct((S0, S1, D), jnp.float32),
        scratch_shapes=[pltpu.SemaphoreType.DMA((_NDMA,))],
    )


def kernel(x, table):
    S0, S1 = x.shape
    V, D = table.shape
    B = S0 * S1
    flat = x.reshape(B).astype(jnp.int32)
    y = _make_sc_gather(B, V, D, 800)(flat, table)
    return _make_tc_relayout(S0, S1, D)(y)


# speedup vs baseline: 11.7263x; 11.7263x over previous
import functools
import math

import jax
import jax.numpy as jnp
from jax import lax
from jax.experimental import pallas as pl
from jax.experimental.pallas import tpu as pltpu
from jax.experimental.pallas import tpu_sc as plsc

_D = 64
_SCALE = 8.0
_LANES = 16
_NDMA = 8


@functools.cache
def _make_sc_gather(B, V, D, chunk):
    NC, NS = 2, 16
    NW = NC * NS
    b_per_w = B // NW
    assert b_per_w * NW == B and b_per_w % chunk == 0
    n_chunks = b_per_w // chunk
    mesh = plsc.VectorSubcoreMesh(core_axis_name="c", subcore_axis_name="s")

    @functools.partial(
        pl.kernel,
        out_type=jax.ShapeDtypeStruct((B, D), jnp.float32),
        mesh=mesh,
        scratch_types=[
            pltpu.VMEM((chunk,), jnp.int32),
            pltpu.VMEM((chunk,), jnp.int32),
            pltpu.VMEM((chunk, D), jnp.float32),
            pltpu.VMEM((chunk, D), jnp.float32),
            pltpu.SemaphoreType.DMA,
            pltpu.SemaphoreType.DMA,
            pltpu.SemaphoreType.DMA,
            pltpu.SemaphoreType.DMA,
        ],
        compiler_params=pltpu.CompilerParams(use_tc_tiling_on_sc=False),
    )
    def sc_gather(x_hbm, table_hbm, out_hbm, idx0, idx1, rows0, rows1,
                  sg0, sg1, ss0, ss1):
        wid = lax.axis_index("s") * NC + lax.axis_index("c")
        base = wid * b_per_w
        slots = ((idx0, rows0, sg0, ss0), (idx1, rows1, sg1, ss1))

        def start_gather(g, slot):
            idx, rows, sg, _ = slot
            pltpu.sync_copy(x_hbm.at[pl.ds(base + g * chunk, chunk)], idx)
            pltpu.async_copy(table_hbm.at[idx], rows, sg)

        def wait_gather(slot):
            idx, rows, sg, _ = slot
            pltpu.make_async_copy(table_hbm.at[idx], rows, sg).wait()

        def scale(slot):
            rows = slot[1]

            def row_body(r, _):
                for j in range(D // _LANES):
                    sl = pl.ds(j * _LANES, _LANES)
                    rows[r, sl] = rows[r, sl] * _SCALE
                return ()

            lax.fori_loop(0, chunk, row_body, (), unroll=8)

        def start_scatter(g, slot):
            _, rows, _, ss = slot
            pltpu.async_copy(rows, out_hbm.at[pl.ds(base + g * chunk, chunk)], ss)

        def wait_scatter(g, slot):
            _, rows, _, ss = slot
            pltpu.make_async_copy(
                rows, out_hbm.at[pl.ds(base + g * chunk, chunk)], ss).wait()

        start_gather(0, slots[0])

        def pair(p, _):
            for b in range(2):
                g = p * 2 + b
                nslot = slots[1 - b]

                @pl.when(g + 1 < n_chunks)
                def _():
                    @pl.when(g >= 1)
                    def _():
                        wait_scatter(g - 1, nslot)

                    start_gather(g + 1, nslot)

                wait_gather(slots[b])
                scale(slots[b])
                start_scatter(g, slots[b])
            return ()

        lax.fori_loop(0, n_chunks // 2, pair, ())
        wait_scatter(n_chunks - 2, slots[0])
        wait_scatter(n_chunks - 1, slots[1])

    return sc_gather


@functools.cache
def _make_tc_relayout(S0, S1, D, G):
    def body(i_ref, o_ref):
        o_ref[...] = i_ref[...].reshape(G, S1, D)

    return pl.pallas_call(
        body,
        grid=(S0 // G,),
        in_specs=[pl.BlockSpec((G * S1, D), lambda i: (i, 0))],
        out_specs=pl.BlockSpec((G, S1, D), lambda i: (i, 0, 0)),
        out_shape=jax.ShapeDtypeStruct((S0, S1, D), jnp.float32),
    )


def kernel(x, table):
    S0, S1 = x.shape
    V, D = table.shape
    B = S0 * S1
    flat = x.reshape(B).astype(jnp.int32)
    y = _make_sc_gather(B, V, D, 800)(flat, table)
    return _make_tc_relayout(S0, S1, D, 64)(y)
